# baseline (speedup 1.0000x reference)
import jax
import jax.numpy as jnp
import numpy as np
from jax.experimental import pallas as pl
from jax.experimental.pallas import tpu as pltpu

NV, NE, NF = 4096, 8192, 4096
BM = 1024
BK = 2048
NSLOT = 4

_G_SHAPES = [(NV, NV), (NE, NV), (NE, NE), (NF, NE), (NF, NF)]
_X_SRC = [0, 0, 1, 1, 2]
_OUT_ID = [0, 1, 1, 2, 2]
_OUT_NBLK = [NV // BM, NE // BM, NF // BM]
_Y_BASE = np.cumsum([0] + [k // BK for (_, k) in _G_SHAPES]).tolist()
_NYCHUNK = _Y_BASE[-1]


def _build_schedule():
    seg, roff, coff, ychunk, firstk, lastk, outid, oblk = \
        [], [], [], [], [], [], [], []
    contributors = [[g for g in range(5) if _OUT_ID[g] == o]
                    for o in range(3)]
    for o in range(3):
        for j in range(_OUT_NBLK[o]):
            runs = []
            for g in contributors[o]:
                n_k = _G_SHAPES[g][1] // BK
                for k in range(n_k):
                    runs.append((g, k))
            for idx, (g, k) in enumerate(runs):
                seg.append(g)
                roff.append(j * BM)
                coff.append(k * BK)
                ychunk.append(_Y_BASE[g] + k)
                firstk.append(1 if idx == 0 else 0)
                lastk.append(1 if idx == len(runs) - 1 else 0)
                outid.append(o)
                oblk.append(j)
    arrs = [seg, roff, coff, ychunk, firstk, lastk, outid, oblk]
    return [np.asarray(a, dtype=np.int32) for a in arrs]


_SCHED = _build_schedule()
_NSTEP = len(_SCHED[0])


def _y_kernel(xv_ref, xe_ref, xf_ref,
              w0_ref, w1_ref, w2_ref, w3_ref, w4_ref, y_ref):
    x_refs = [xv_ref, xe_ref, xf_ref]
    w_refs = [w0_ref, w1_ref, w2_ref, w3_ref, w4_ref]
    for g in range(5):
        x_ref = x_refs[_X_SRC[g]]
        w16 = w_refs[g][...].astype(jnp.bfloat16)
        n_k = _G_SHAPES[g][1] // BK
        for k in range(n_k):
            x_blk = x_ref[pl.ds(k * BK, BK), :].astype(jnp.bfloat16)
            y_ref[_Y_BASE[g] + k] = jax.lax.dot_general(
                w16, x_blk,
                dimension_numbers=(((0,), (1,)), ((), ())),
                preferred_element_type=jnp.float32).astype(jnp.bfloat16)


def _big_kernel(seg_ref, roff_ref, coff_ref, ychunk_ref, fk_ref,
                lk_ref, oid_ref, oblk_ref,
                y_ref, g0_ref, g1_ref, g2_ref, g3_ref, g4_ref,
                ov_ref, oe_ref, of_ref, t_ref, buf_ref, sem_ref):
    s = pl.program_id(0)
    g_refs = [g0_ref, g1_ref, g2_ref, g3_ref, g4_ref]
    o_refs = [ov_ref, oe_ref, of_ref]

    def enqueue(t, slot):
        half = BM // 2
        for c in range(5):
            @pl.when(seg_ref[t] == c)
            def _(c=c):
                r0 = pl.multiple_of(roff_ref[t], BM)
                c0 = pl.multiple_of(coff_ref[t], BK)
                src_lo = g_refs[c].at[pl.ds(r0, half), pl.ds(c0, BK)]
                src_hi = g_refs[c].at[pl.ds(r0 + half, half), pl.ds(c0, BK)]
                pltpu.make_async_copy(
                    src_lo, buf_ref.at[slot, pl.ds(0, half)],
                    sem_ref.at[slot]).start()
                pltpu.make_async_copy(
                    src_hi, buf_ref.at[slot, pl.ds(half, half)],
                    sem_ref.at[slot]).start()

    @pl.when(s == 0)
    def _():
        for j in range(NSLOT):
            enqueue(j, j)

    slot = jax.lax.rem(s, NSLOT)
    half = BM // 2
    for h in range(2):
        pltpu.make_async_copy(
            g0_ref.at[pl.ds(h * half, half), pl.ds(0, BK)],
            buf_ref.at[slot, pl.ds(h * half, half)],
            sem_ref.at[slot]).wait()

    g16 = buf_ref[slot].astype(jnp.bfloat16)
    y_blk = y_ref[ychunk_ref[s]]
    part = jax.lax.dot_general(
        y_blk, g16,
        dimension_numbers=(((1,), (1,)), ((), ())),
        preferred_element_type=jnp.float32)

    acc = jnp.where(fk_ref[s] == 1, jnp.zeros_like(part), t_ref[...]) + part
    t_ref[...] = acc

    @pl.when(lk_ref[s] == 1)
    def _():
        j = oblk_ref[s]
        for oid in range(3):
            @pl.when(oid_ref[s] == oid)
            def _(oid=oid):
                o_refs[oid][j] = acc

    @pl.when(s + NSLOT < _NSTEP)
    def _():
        enqueue(s + NSLOT, slot)


@jax.jit
def kernel(xv, xe, xf, Gv2v, Gv2e, Ge2e, Ge2f, Gf2f, Wv2v, Wve, Wee, Wef, Wff):
    yt = pl.pallas_call(
        _y_kernel,
        out_shape=jax.ShapeDtypeStruct((_NYCHUNK, 32, BK), jnp.bfloat16),
    )(xv, xe, xf, Wv2v, Wve, Wee, Wef, Wff)

    hbm_spec = pl.BlockSpec(memory_space=pltpu.MemorySpace.HBM)
    ovt, oet, oft = pl.pallas_call(
        _big_kernel,
        grid_spec=pltpu.PrefetchScalarGridSpec(
            num_scalar_prefetch=8,
            grid=(_NSTEP,),
            in_specs=[
                pl.BlockSpec((_NYCHUNK, 32, BK), lambda s, *_: (0, 0, 0)),
                hbm_spec, hbm_spec, hbm_spec, hbm_spec, hbm_spec,
            ],
            out_specs=(
                pl.BlockSpec((NV // BM, 32, BM), lambda s, *_: (0, 0, 0)),
                pl.BlockSpec((NE // BM, 32, BM), lambda s, *_: (0, 0, 0)),
                pl.BlockSpec((NF // BM, 32, BM), lambda s, *_: (0, 0, 0)),
            ),
            scratch_shapes=[
                pltpu.VMEM((32, BM), jnp.float32),
                pltpu.VMEM((NSLOT, BM, BK), jnp.float32),
                pltpu.SemaphoreType.DMA((NSLOT,)),
            ],
        ),
        out_shape=(
            jax.ShapeDtypeStruct((NV // BM, 32, BM), jnp.float32),
            jax.ShapeDtypeStruct((NE // BM, 32, BM), jnp.float32),
            jax.ShapeDtypeStruct((NF // BM, 32, BM), jnp.float32),
        ),
        compiler_params=pltpu.CompilerParams(
            dimension_semantics=("arbitrary",),
        ),
    )(*_SCHED, yt, Gv2v, Gv2e, Ge2e, Ge2f, Gf2f)

    zv = ovt.transpose(0, 2, 1).reshape(NV, 32)
    ze = oet.transpose(0, 2, 1).reshape(NE, 32)
    zf = oft.transpose(0, 2, 1).reshape(NF, 32)
    return (zv, ze, zf)


# speedup vs baseline: 1.0006x; 1.0006x over previous
import jax
import jax.numpy as jnp
import numpy as np
from jax.experimental import pallas as pl
from jax.experimental.pallas import tpu as pltpu

NV, NE, NF = 4096, 8192, 4096
BM = 1024
BK = 2048
NSLOT = 4

_G_SHAPES = [(NV, NV), (NE, NV), (NE, NE), (NF, NE), (NF, NF)]
_X_SRC = [0, 0, 1, 1, 2]
_OUT_ID = [0, 1, 1, 2, 2]
_OUT_NBLK = [NV // BM, NE // BM, NF // BM]
_Y_BASE = np.cumsum([0] + [k // BK for (_, k) in _G_SHAPES]).tolist()
_NYCHUNK = _Y_BASE[-1]


_OUT_ADD = [0, 0, 1, 0, 1]


def _build_schedule():
    seg, roff, coff, ychunk, firstk, lastk, outid, oblk, oadd = \
        [], [], [], [], [], [], [], [], []
    for g, (m, kdim) in enumerate(_G_SHAPES):
        n_i, n_k = m // BM, kdim // BK
        for i in range(n_i):
            for k in range(n_k):
                seg.append(g)
                roff.append(i * BM)
                coff.append(k * BK)
                ychunk.append(_Y_BASE[g] + k)
                firstk.append(1 if k == 0 else 0)
                lastk.append(1 if k == n_k - 1 else 0)
                outid.append(_OUT_ID[g])
                oblk.append(i)
                oadd.append(_OUT_ADD[g])
    arrs = [seg, roff, coff, ychunk, firstk, lastk, outid, oblk, oadd]
    return [np.asarray(a, dtype=np.int32) for a in arrs]


_SCHED = _build_schedule()
_NSTEP = len(_SCHED[0])


def _y_kernel(xv_ref, xe_ref, xf_ref,
              w0_ref, w1_ref, w2_ref, w3_ref, w4_ref, y_ref):
    x_refs = [xv_ref, xe_ref, xf_ref]
    w_refs = [w0_ref, w1_ref, w2_ref, w3_ref, w4_ref]
    for g in range(5):
        x_ref = x_refs[_X_SRC[g]]
        w16 = w_refs[g][...].astype(jnp.bfloat16)
        n_k = _G_SHAPES[g][1] // BK
        for k in range(n_k):
            x_blk = x_ref[pl.ds(k * BK, BK), :].astype(jnp.bfloat16)
            y_ref[_Y_BASE[g] + k] = jax.lax.dot_general(
                w16, x_blk,
                dimension_numbers=(((0,), (1,)), ((), ())),
                preferred_element_type=jnp.float32).astype(jnp.bfloat16)


def _big_kernel(seg_ref, roff_ref, coff_ref, ychunk_ref, fk_ref,
                lk_ref, oid_ref, oblk_ref, oadd_ref,
                y_ref, g0_ref, g1_ref, g2_ref, g3_ref, g4_ref,
                ov_ref, oe_ref, of_ref, t_ref, buf_ref, sem_ref):
    s = pl.program_id(0)
    g_refs = [g0_ref, g1_ref, g2_ref, g3_ref, g4_ref]
    o_refs = [ov_ref, oe_ref, of_ref]

    def enqueue(t, slot):
        half = BM // 2
        for c in range(5):
            @pl.when(seg_ref[t] == c)
            def _(c=c):
                r0 = pl.multiple_of(roff_ref[t], BM)
                c0 = pl.multiple_of(coff_ref[t], BK)
                src_lo = g_refs[c].at[pl.ds(r0, half), pl.ds(c0, BK)]
                src_hi = g_refs[c].at[pl.ds(r0 + half, half), pl.ds(c0, BK)]
                pltpu.make_async_copy(
                    src_lo, buf_ref.at[slot, pl.ds(0, half)],
                    sem_ref.at[slot]).start()
                pltpu.make_async_copy(
                    src_hi, buf_ref.at[slot, pl.ds(half, half)],
                    sem_ref.at[slot]).start()

    @pl.when(s == 0)
    def _():
        for j in range(NSLOT):
            enqueue(j, j)

    slot = jax.lax.rem(s, NSLOT)
    half = BM // 2
    for h in range(2):
        pltpu.make_async_copy(
            g0_ref.at[pl.ds(h * half, half), pl.ds(0, BK)],
            buf_ref.at[slot, pl.ds(h * half, half)],
            sem_ref.at[slot]).wait()

    g16 = buf_ref[slot].astype(jnp.bfloat16)
    y_blk = y_ref[ychunk_ref[s]]
    part = jax.lax.dot_general(
        y_blk, g16,
        dimension_numbers=(((1,), (1,)), ((), ())),
        preferred_element_type=jnp.float32)

    acc = jnp.where(fk_ref[s] == 1, jnp.zeros_like(part), t_ref[...]) + part
    t_ref[...] = acc

    @pl.when(lk_ref[s] == 1)
    def _():
        j = oblk_ref[s]
        for oid in range(3):
            @pl.when(oid_ref[s] == oid)
            def _(oid=oid):
                prev = jnp.where(oadd_ref[s] == 1, o_refs[oid][j],
                                 jnp.zeros_like(acc))
                o_refs[oid][j] = prev + acc

    @pl.when(s + NSLOT < _NSTEP)
    def _():
        enqueue(s + NSLOT, slot)


@jax.jit
def kernel(xv, xe, xf, Gv2v, Gv2e, Ge2e, Ge2f, Gf2f, Wv2v, Wve, Wee, Wef, Wff):
    yt = pl.pallas_call(
        _y_kernel,
        out_shape=jax.ShapeDtypeStruct((_NYCHUNK, 32, BK), jnp.bfloat16),
    )(xv, xe, xf, Wv2v, Wve, Wee, Wef, Wff)

    hbm_spec = pl.BlockSpec(memory_space=pltpu.MemorySpace.HBM)
    ovt, oet, oft = pl.pallas_call(
        _big_kernel,
        grid_spec=pltpu.PrefetchScalarGridSpec(
            num_scalar_prefetch=9,
            grid=(_NSTEP,),
            in_specs=[
                pl.BlockSpec((_NYCHUNK, 32, BK), lambda s, *_: (0, 0, 0)),
                hbm_spec, hbm_spec, hbm_spec, hbm_spec, hbm_spec,
            ],
            out_specs=(
                pl.BlockSpec((NV // BM, 32, BM), lambda s, *_: (0, 0, 0)),
                pl.BlockSpec((NE // BM, 32, BM), lambda s, *_: (0, 0, 0)),
                pl.BlockSpec((NF // BM, 32, BM), lambda s, *_: (0, 0, 0)),
            ),
            scratch_shapes=[
                pltpu.VMEM((32, BM), jnp.float32),
                pltpu.VMEM((NSLOT, BM, BK), jnp.float32),
                pltpu.SemaphoreType.DMA((NSLOT,)),
            ],
        ),
        out_shape=(
            jax.ShapeDtypeStruct((NV // BM, 32, BM), jnp.float32),
            jax.ShapeDtypeStruct((NE // BM, 32, BM), jnp.float32),
            jax.ShapeDtypeStruct((NF // BM, 32, BM), jnp.float32),
        ),
        compiler_params=pltpu.CompilerParams(
            dimension_semantics=("arbitrary",),
        ),
    )(*_SCHED, yt, Gv2v, Gv2e, Ge2e, Ge2f, Gf2f)

    zv = ovt.transpose(0, 2, 1).reshape(NV, 32)
    ze = oet.transpose(0, 2, 1).reshape(NE, 32)
    zf = oft.transpose(0, 2, 1).reshape(NF, 32)
    return (zv, ze, zf)


# speedup vs baseline: 1.0553x; 1.0547x over previous
import jax
import jax.numpy as jnp
import numpy as np
from jax.experimental import pallas as pl
from jax.experimental.pallas import tpu as pltpu

NV, NE, NF = 4096, 8192, 4096
BM = 1024
BK = 2048
NSLOT = 4

_G_SHAPES = [(NV, NV), (NE, NV), (NE, NE), (NF, NE), (NF, NF)]
_X_OFF = [0, 0, NV, NV, NV + NE]


def _build_schedule():
    seg, roff, coff, blk, xrow, firstk = [], [], [], [], [], []
    blk_base = 0
    for g, (m, kdim) in enumerate(_G_SHAPES):
        n_i, n_k = m // BM, kdim // BK
        for i in range(n_i):
            for k in range(n_k):
                seg.append(g)
                roff.append(i * BM)
                coff.append(k * BK)
                blk.append(blk_base + i)
                xrow.append(_X_OFF[g] + k * BK)
                firstk.append(1 if k == 0 else 0)
        blk_base += n_i
    arrs = [seg, roff, coff, blk, xrow, firstk]
    return [np.asarray(a, dtype=np.int32) for a in arrs], blk_base


_SCHED, _NBLK = _build_schedule()
_NSTEP = len(_SCHED[0])


def _big_kernel(seg_ref, roff_ref, coff_ref, blk_ref, xrow_ref, fk_ref,
                xall_ref, g0_ref, g1_ref, g2_ref, g3_ref, g4_ref,
                t_ref, buf_ref, sem_ref):
    s = pl.program_id(0)
    g_refs = [g0_ref, g1_ref, g2_ref, g3_ref, g4_ref]

    def enqueue(t, slot):
        half = BM // 2
        for c in range(5):
            @pl.when(seg_ref[t] == c)
            def _(c=c):
                r0 = pl.multiple_of(roff_ref[t], BM)
                c0 = pl.multiple_of(coff_ref[t], BK)
                src_lo = g_refs[c].at[pl.ds(r0, half), pl.ds(c0, BK)]
                src_hi = g_refs[c].at[pl.ds(r0 + half, half), pl.ds(c0, BK)]
                pltpu.make_async_copy(
                    src_lo, buf_ref.at[slot, pl.ds(0, half)],
                    sem_ref.at[slot]).start()
                pltpu.make_async_copy(
                    src_hi, buf_ref.at[slot, pl.ds(half, half)],
                    sem_ref.at[slot]).start()

    @pl.when(s == 0)
    def _():
        for j in range(NSLOT):
            enqueue(j, j)

    slot = jax.lax.rem(s, NSLOT)
    half = BM // 2
    for h in range(2):
        pltpu.make_async_copy(
            g0_ref.at[pl.ds(h * half, half), pl.ds(0, BK)],
            buf_ref.at[slot, pl.ds(h * half, half)],
            sem_ref.at[slot]).wait()

    g16 = buf_ref[slot].astype(jnp.bfloat16)
    x_blk = xall_ref[pl.ds(pl.multiple_of(xrow_ref[s], BK), BK), :]
    part = jax.lax.dot_general(
        x_blk, g16,
        dimension_numbers=(((0,), (1,)), ((), ())),
        preferred_element_type=jnp.float32)

    b = blk_ref[s]
    prev = jnp.where(fk_ref[s] == 1, jnp.zeros_like(part), t_ref[b])
    t_ref[b] = prev + part

    @pl.when(s + NSLOT < _NSTEP)
    def _():
        enqueue(s + NSLOT, slot)


def _w_apply_kernel(t_ref, wv_ref, we1_ref, we2_ref, wf1_ref, wf2_ref,
                    ov_ref, oe_ref, of_ref):
    def wt(w_ref, c):
        return jax.lax.dot_general(
            w_ref[...].astype(jnp.bfloat16),
            t_ref[c].astype(jnp.bfloat16),
            dimension_numbers=(((0,), (0,)), ((), ())),
            preferred_element_type=jnp.float32)

    nv_b, ne_b, nf_b = NV // BM, NE // BM, NF // BM
    o = 0
    for j in range(nv_b):
        ov_ref[:, pl.ds(j * BM, BM)] = wt(wv_ref, o + j)
    o += nv_b
    for j in range(ne_b):
        oe_ref[:, pl.ds(j * BM, BM)] = (wt(we1_ref, o + j)
                                        + wt(we2_ref, o + ne_b + j))
    o += 2 * ne_b
    for j in range(nf_b):
        of_ref[:, pl.ds(j * BM, BM)] = (wt(wf1_ref, o + j)
                                        + wt(wf2_ref, o + nf_b + j))


@jax.jit
def kernel(xv, xe, xf, Gv2v, Gv2e, Ge2e, Ge2f, Gf2f, Wv2v, Wve, Wee, Wef, Wff):
    xall = jnp.concatenate([xv, xe, xf], axis=0).astype(jnp.bfloat16)

    hbm_spec = pl.BlockSpec(memory_space=pltpu.MemorySpace.HBM)
    t_all = pl.pallas_call(
        _big_kernel,
        grid_spec=pltpu.PrefetchScalarGridSpec(
            num_scalar_prefetch=6,
            grid=(_NSTEP,),
            in_specs=[
                pl.BlockSpec((NV + NE + NF, 32), lambda s, *_: (0, 0)),
                hbm_spec, hbm_spec, hbm_spec, hbm_spec, hbm_spec,
            ],
            out_specs=pl.BlockSpec((_NBLK, 32, BM), lambda s, *_: (0, 0, 0)),
            scratch_shapes=[
                pltpu.VMEM((NSLOT, BM, BK), jnp.float32),
                pltpu.SemaphoreType.DMA((NSLOT,)),
            ],
        ),
        out_shape=jax.ShapeDtypeStruct((_NBLK, 32, BM), jnp.float32),
        compiler_params=pltpu.CompilerParams(
            dimension_semantics=("arbitrary",),
        ),
    )(*_SCHED, xall, Gv2v, Gv2e, Ge2e, Ge2f, Gf2f)

    zvt, zet, zft = pl.pallas_call(
        _w_apply_kernel,
        out_shape=(
            jax.ShapeDtypeStruct((32, NV), jnp.float32),
            jax.ShapeDtypeStruct((32, NE), jnp.float32),
            jax.ShapeDtypeStruct((32, NF), jnp.float32),
        ),
    )(t_all, Wv2v, Wve, Wee, Wef, Wff)
    return (zvt.T, zet.T, zft.T)
